# Initial kernel scaffold; baseline (speedup 1.0000x reference)
import jax
import jax.numpy as jnp
from jax.experimental import pallas as pl


def kernel(idx_of_objs, valid2all, syb_graph, att_weights, vis_len):
    raise NotImplementedError("write your pallas kernel here")


# speedup vs baseline: 7.7067x; 7.7067x over previous
import jax
import jax.numpy as jnp
from jax.experimental import pallas as pl

MARGIN = 0.6


def _body(val_ref, idxo_ref, syb_ref, att_ref, out_ref):
    blocks = att_ref.shape[0]
    v = syb_ref.shape[1]

    vrow = val_ref[0]
    iorow = idxo_ref[0]

    citer = jax.lax.broadcasted_iota(jnp.int32, (v, v), 0)
    ovt = (jnp.broadcast_to(vrow, (v, v)) == citer).astype(jnp.float32)
    oit = (jnp.broadcast_to(iorow, (v, v)) == citer).astype(jnp.float32)

    eq = jax.lax.dot_general(ovt, ovt, (((0,), (0,)), ((), ())),
                             preferred_element_type=jnp.float32)
    lprime = jax.lax.broadcasted_iota(jnp.int32, (v, v), 1)
    first = jnp.min(jnp.where(eq > 0.5, lprime, v), axis=1,
                    keepdims=True)
    lidx = jax.lax.broadcasted_iota(jnp.int32, (v, 1), 0)
    m = (first == lidx).astype(jnp.float32)

    sybsign = 1.0 - 2.0 * syb_ref[0].astype(jnp.float32)
    gs = jax.lax.dot_general(oit, sybsign, (((0,), (0,)), ((), ())),
                             preferred_element_type=jnp.float32)

    partial = jnp.float32(0.0)
    for b in range(blocks):
        d = jnp.sum(att_ref[b, 0] * gs, axis=1, keepdims=True)
        partial += jnp.sum(jnp.maximum(d + MARGIN, 0.0) * m)
    nfirst = jnp.sum(m)
    partial += blocks * (v - nfirst) * MARGIN

    tile = jnp.full((8, 128), partial, dtype=jnp.float32)
    i = pl.program_id(0)

    @pl.when(i == 0)
    def _init():
        out_ref[...] = tile

    @pl.when(i > 0)
    def _acc():
        out_ref[...] += tile


def kernel(idx_of_objs, valid2all, syb_graph, att_weights, vis_len):
    del vis_len
    blocks, bsz, v, s = att_weights.shape
    val3 = valid2all.reshape(bsz, 1, v)
    idx3 = idx_of_objs.reshape(bsz, 1, v)

    out = pl.pallas_call(
        _body,
        grid=(bsz,),
        in_specs=[
            pl.BlockSpec((1, 1, v), lambda i: (i, 0, 0)),
            pl.BlockSpec((1, 1, v), lambda i: (i, 0, 0)),
            pl.BlockSpec((1, v, s), lambda i: (i, 0, 0)),
            pl.BlockSpec((blocks, 1, v, s), lambda i: (0, i, 0, 0)),
        ],
        out_specs=pl.BlockSpec((8, 128), lambda i: (0, 0)),
        out_shape=jax.ShapeDtypeStruct((8, 128), jnp.float32),
    )(val3, idx3, syb_graph, att_weights)

    total = jnp.float32(blocks * bsz * v)
    return out[0, 0] / total
